# baseline (speedup 1.0000x reference)
import functools

import jax
import jax.numpy as jnp
from jax.experimental import pallas as pl
from jax.experimental.pallas import tpu as pltpu

_N_ROWS = 128
_N_COLS = 100000

_SUB = 128
_GROUP = 8
_CHUNK = _SUB * _GROUP
_NFULL = _N_COLS // _CHUNK
_TAIL_START = _NFULL * _CHUNK
_TAIL_WIDTHS = [_SUB] * 5 + [32]

_ROT_A = (13, 15, 26, 6)
_ROT_B = (17, 29, 16, 24)
_KS = (0, 1, 0x1BD11BDB)


def _rotl(x, r):
    return (x << jnp.uint32(r)) | (x >> jnp.uint32(32 - r))


def _threefry_bits(j):
    ks = tuple(jnp.uint32(k) for k in _KS)
    x1 = j + ks[1]
    x0 = x1
    x1 = _rotl(x1, _ROT_A[0]) ^ x0
    for r in _ROT_A[1:]:
        x0 = x0 + x1
        x1 = _rotl(x1, r)
        x1 = x0 ^ x1
    inject = ((ks[1], ks[2], 1), (ks[2], ks[0], 2), (ks[0], ks[1], 3),
              (ks[1], ks[2], 4), (ks[2], ks[0], 5))
    rots = (_ROT_B, _ROT_A, _ROT_B, _ROT_A)
    for (ka, kb, c), rgroup in zip(inject, rots + ((),)):
        x0 = x0 + ka
        x1 = x1 + kb + jnp.uint32(c)
        for r in rgroup:
            x0 = x0 + x1
            x1 = _rotl(x1, r)
            x1 = x0 ^ x1
    return x0 ^ x1


def _z_sub(x, idx, rt):
    bits = _threefry_bits(idx)
    f = jax.lax.bitcast_convert_type(
        (bits >> jnp.uint32(9)) | jnp.uint32(0x3F800000), jnp.float32)
    u = f - jnp.float32(1.0)
    eps = jnp.float32(1e-20)
    g = -jnp.log(-jnp.log(u + eps) + eps)
    return (x + g) * rt


def _body(x_ref, t_ref, o_ref, *, block_rows):
    i = pl.program_id(0)
    rt = jnp.float32(1.0) / t_ref[0]
    shape = (block_rows, _SUB)
    row = jax.lax.broadcasted_iota(jnp.uint32, shape, 0)
    col = jax.lax.broadcasted_iota(jnp.uint32, shape, 1)
    base_row = (i * block_rows).astype(jnp.uint32)
    idx0 = (base_row + row) * jnp.uint32(_N_COLS) + col

    def p1(k, m):
        off = k * _CHUNK
        for j in range(_GROUP):
            o = off + j * _SUB
            z = _z_sub(x_ref[:, pl.ds(o, _SUB)],
                       idx0 + o.astype(jnp.uint32), rt)
            o_ref[:, pl.ds(o, _SUB)] = z
            m = jnp.maximum(m, z)
        return m

    m_lanes = jax.lax.fori_loop(
        0, _NFULL, p1, jnp.full(shape, -jnp.inf, jnp.float32))

    z_tails = []
    o = _TAIL_START
    for w in _TAIL_WIDTHS:
        z = _z_sub(x_ref[:, o:o + w],
                   idx0[:, :w] + jnp.uint32(o), rt)
        o_ref[:, o:o + w] = z
        z_tails.append(z)
        o += w
    m = jnp.max(m_lanes, axis=-1, keepdims=True)
    for z in z_tails:
        m = jnp.maximum(m, jnp.max(z, axis=-1, keepdims=True))

    def p2(k, s):
        off = k * _CHUNK
        for j in range(_GROUP):
            o = off + j * _SUB
            e = jnp.exp(o_ref[:, pl.ds(o, _SUB)] - m)
            o_ref[:, pl.ds(o, _SUB)] = e
            s = s + e
        return s

    s_lanes = jax.lax.fori_loop(0, _NFULL, p2, jnp.zeros(shape, jnp.float32))
    s = jnp.sum(s_lanes, axis=-1, keepdims=True)
    e_tails = []
    o = _TAIL_START
    for z, w in zip(z_tails, _TAIL_WIDTHS):
        e = jnp.exp(z - m)
        o_ref[:, o:o + w] = e
        e_tails.append(e)
        s = s + jnp.sum(e, axis=-1, keepdims=True)
        o += w
    rs = jnp.float32(1.0) / s

    def p3(k, carry):
        off = k * _CHUNK
        for j in range(_GROUP):
            o = off + j * _SUB
            o_ref[:, pl.ds(o, _SUB)] = o_ref[:, pl.ds(o, _SUB)] * rs
        return carry

    jax.lax.fori_loop(0, _NFULL, p3, 0)
    o = _TAIL_START
    for e, w in zip(e_tails, _TAIL_WIDTHS):
        o_ref[:, o:o + w] = e * rs
        o += w


def kernel(logits, temperature, use_gpu):
    del use_gpu
    block_rows = 8
    t = jnp.float32(temperature).reshape(1)
    return pl.pallas_call(
        functools.partial(_body, block_rows=block_rows),
        grid=(_N_ROWS // block_rows,),
        in_specs=[
            pl.BlockSpec((block_rows, _N_COLS), lambda i: (i, 0)),
            pl.BlockSpec(memory_space=pltpu.SMEM),
        ],
        out_specs=pl.BlockSpec((block_rows, _N_COLS), lambda i: (i, 0)),
        out_shape=jax.ShapeDtypeStruct((_N_ROWS, _N_COLS), jnp.float32),
    )(logits, t)


# speedup vs baseline: 1.4273x; 1.4273x over previous
import functools

import jax
import jax.numpy as jnp
from jax.experimental import pallas as pl
from jax.experimental.pallas import tpu as pltpu

_N_ROWS = 128
_N_COLS = 100000

_CHUNK = 2048
_NFULL = _N_COLS // _CHUNK
_TAIL_START = _NFULL * _CHUNK
_TAIL = _N_COLS - _TAIL_START

_ROT_A = (13, 15, 26, 6)
_ROT_B = (17, 29, 16, 24)
_KS = (0, 1, 0x1BD11BDB)


def _rotl(x, r):
    return (x << jnp.uint32(r)) | (x >> jnp.uint32(32 - r))


def _threefry_bits(j):
    ks = tuple(jnp.uint32(k) for k in _KS)
    x1 = j + ks[1]
    x0 = x1
    x1 = _rotl(x1, _ROT_A[0]) ^ x0
    for r in _ROT_A[1:]:
        x0 = x0 + x1
        x1 = _rotl(x1, r)
        x1 = x0 ^ x1
    inject = ((ks[1], ks[2], 1), (ks[2], ks[0], 2), (ks[0], ks[1], 3),
              (ks[1], ks[2], 4), (ks[2], ks[0], 5))
    rots = (_ROT_B, _ROT_A, _ROT_B, _ROT_A)
    for (ka, kb, c), rgroup in zip(inject, rots + ((),)):
        x0 = x0 + ka
        x1 = x1 + kb + jnp.uint32(c)
        for r in rgroup:
            x0 = x0 + x1
            x1 = _rotl(x1, r)
            x1 = x0 ^ x1
    return x0 ^ x1


def _z_chunk(x, idx, rt):
    bits = _threefry_bits(idx)
    f = jax.lax.bitcast_convert_type(
        (bits >> jnp.uint32(9)) | jnp.uint32(0x3F800000), jnp.float32)
    u = f - jnp.float32(1.0)
    eps = jnp.float32(1e-20)
    g = -jnp.log(-jnp.log(u + eps) + eps)
    return (x + g) * rt


def _body(x_ref, t_ref, o_ref, *, block_rows):
    i = pl.program_id(0)
    rt = jnp.float32(1.0) / t_ref[0]
    shape = (block_rows, _CHUNK)
    row = jax.lax.broadcasted_iota(jnp.uint32, shape, 0)
    col = jax.lax.broadcasted_iota(jnp.uint32, shape, 1)
    base_row = (i * block_rows).astype(jnp.uint32)
    idx0 = (base_row + row) * jnp.uint32(_N_COLS) + col

    def p1(k, m):
        off = k * _CHUNK
        z = _z_chunk(x_ref[:, pl.ds(off, _CHUNK)],
                     idx0 + off.astype(jnp.uint32), rt)
        o_ref[:, pl.ds(off, _CHUNK)] = z
        return jnp.maximum(m, z)

    m_lanes = jax.lax.fori_loop(
        0, _NFULL, p1, jnp.full(shape, -jnp.inf, jnp.float32))
    z_tail = _z_chunk(x_ref[:, _TAIL_START:],
                      idx0[:, :_TAIL] + jnp.uint32(_TAIL_START), rt)
    o_ref[:, _TAIL_START:] = z_tail
    m = jnp.maximum(jnp.max(m_lanes, axis=-1, keepdims=True),
                    jnp.max(z_tail, axis=-1, keepdims=True))

    def p2(k, s):
        off = k * _CHUNK
        e = jnp.exp(o_ref[:, pl.ds(off, _CHUNK)] - m)
        o_ref[:, pl.ds(off, _CHUNK)] = e
        return s + e

    s_lanes = jax.lax.fori_loop(0, _NFULL, p2, jnp.zeros(shape, jnp.float32))
    e_tail = jnp.exp(z_tail - m)
    o_ref[:, _TAIL_START:] = e_tail
    s = (jnp.sum(s_lanes, axis=-1, keepdims=True)
         + jnp.sum(e_tail, axis=-1, keepdims=True))
    rs = jnp.float32(1.0) / s

    def p3(k, carry):
        off = k * _CHUNK
        o_ref[:, pl.ds(off, _CHUNK)] = o_ref[:, pl.ds(off, _CHUNK)] * rs
        return carry

    jax.lax.fori_loop(0, _NFULL, p3, 0)
    o_ref[:, _TAIL_START:] = e_tail * rs


def kernel(logits, temperature, use_gpu):
    del use_gpu
    block_rows = 8
    t = jnp.float32(temperature).reshape(1)
    return pl.pallas_call(
        functools.partial(_body, block_rows=block_rows),
        grid=(_N_ROWS // block_rows,),
        in_specs=[
            pl.BlockSpec((block_rows, _N_COLS), lambda i: (i, 0)),
            pl.BlockSpec(memory_space=pltpu.SMEM),
        ],
        out_specs=pl.BlockSpec((block_rows, _N_COLS), lambda i: (i, 0)),
        out_shape=jax.ShapeDtypeStruct((_N_ROWS, _N_COLS), jnp.float32),
    )(logits, t)


# speedup vs baseline: 1.5273x; 1.0700x over previous
import functools

import jax
import jax.numpy as jnp
from jax.experimental import pallas as pl
from jax.experimental.pallas import tpu as pltpu

_N_ROWS = 128
_N_COLS = 100000

_CHUNK = 4096
_NFULL = _N_COLS // _CHUNK
_TAIL_START = _NFULL * _CHUNK
_TAIL = _N_COLS - _TAIL_START

_ROT_A = (13, 15, 26, 6)
_ROT_B = (17, 29, 16, 24)
_KS = (0, 1, 0x1BD11BDB)


def _rotl(x, r):
    return (x << jnp.uint32(r)) | (x >> jnp.uint32(32 - r))


def _threefry_bits(j):
    ks = tuple(jnp.uint32(k) for k in _KS)
    x1 = j + ks[1]
    x0 = x1
    x1 = _rotl(x1, _ROT_A[0]) ^ x0
    for r in _ROT_A[1:]:
        x0 = x0 + x1
        x1 = _rotl(x1, r)
        x1 = x0 ^ x1
    inject = ((ks[1], ks[2], 1), (ks[2], ks[0], 2), (ks[0], ks[1], 3),
              (ks[1], ks[2], 4), (ks[2], ks[0], 5))
    rots = (_ROT_B, _ROT_A, _ROT_B, _ROT_A)
    for (ka, kb, c), rgroup in zip(inject, rots + ((),)):
        x0 = x0 + ka
        x1 = x1 + kb + jnp.uint32(c)
        for r in rgroup:
            x0 = x0 + x1
            x1 = _rotl(x1, r)
            x1 = x0 ^ x1
    return x0 ^ x1


def _z_chunk(x, idx, rt):
    bits = _threefry_bits(idx)
    f = jax.lax.bitcast_convert_type(
        (bits >> jnp.uint32(9)) | jnp.uint32(0x3F800000), jnp.float32)
    u = f - jnp.float32(1.0)
    eps = jnp.float32(1e-20)
    g = -jnp.log(-jnp.log(u + eps) + eps)
    return (x + g) * rt


def _body(x_ref, t_ref, o_ref, *, block_rows):
    i = pl.program_id(0)
    rt = jnp.float32(1.0) / t_ref[0]
    shape = (block_rows, _CHUNK)
    row = jax.lax.broadcasted_iota(jnp.uint32, shape, 0)
    col = jax.lax.broadcasted_iota(jnp.uint32, shape, 1)
    base_row = (i * block_rows).astype(jnp.uint32)
    idx0 = (base_row + row) * jnp.uint32(_N_COLS) + col

    def p1(k, m):
        off = k * _CHUNK
        z = _z_chunk(x_ref[:, pl.ds(off, _CHUNK)],
                     idx0 + off.astype(jnp.uint32), rt)
        o_ref[:, pl.ds(off, _CHUNK)] = z
        return jnp.maximum(m, z)

    m_lanes = jax.lax.fori_loop(
        0, _NFULL, p1, jnp.full(shape, -jnp.inf, jnp.float32))
    z_tail = _z_chunk(x_ref[:, _TAIL_START:],
                      idx0[:, :_TAIL] + jnp.uint32(_TAIL_START), rt)
    o_ref[:, _TAIL_START:] = z_tail
    m = jnp.maximum(jnp.max(m_lanes, axis=-1, keepdims=True),
                    jnp.max(z_tail, axis=-1, keepdims=True))

    def p2(k, s):
        off = k * _CHUNK
        e = jnp.exp(o_ref[:, pl.ds(off, _CHUNK)] - m)
        o_ref[:, pl.ds(off, _CHUNK)] = e
        return s + e

    s_lanes = jax.lax.fori_loop(0, _NFULL, p2, jnp.zeros(shape, jnp.float32))
    e_tail = jnp.exp(z_tail - m)
    o_ref[:, _TAIL_START:] = e_tail
    s = (jnp.sum(s_lanes, axis=-1, keepdims=True)
         + jnp.sum(e_tail, axis=-1, keepdims=True))
    rs = jnp.float32(1.0) / s

    def p3(k, carry):
        off = k * _CHUNK
        o_ref[:, pl.ds(off, _CHUNK)] = o_ref[:, pl.ds(off, _CHUNK)] * rs
        return carry

    jax.lax.fori_loop(0, _NFULL, p3, 0)
    o_ref[:, _TAIL_START:] = e_tail * rs


def kernel(logits, temperature, use_gpu):
    del use_gpu
    block_rows = 8
    t = jnp.float32(temperature).reshape(1)
    return pl.pallas_call(
        functools.partial(_body, block_rows=block_rows),
        grid=(_N_ROWS // block_rows,),
        in_specs=[
            pl.BlockSpec((block_rows, _N_COLS), lambda i: (i, 0)),
            pl.BlockSpec(memory_space=pltpu.SMEM),
        ],
        out_specs=pl.BlockSpec((block_rows, _N_COLS), lambda i: (i, 0)),
        out_shape=jax.ShapeDtypeStruct((_N_ROWS, _N_COLS), jnp.float32),
    )(logits, t)


# speedup vs baseline: 1.5732x; 1.0301x over previous
import functools

import jax
import jax.numpy as jnp
from jax.experimental import pallas as pl
from jax.experimental.pallas import tpu as pltpu

_N_ROWS = 128
_N_COLS = 100000

_CHUNK = 8192
_NFULL = _N_COLS // _CHUNK
_TAIL_START = _NFULL * _CHUNK
_TAIL = _N_COLS - _TAIL_START

_ROT_A = (13, 15, 26, 6)
_ROT_B = (17, 29, 16, 24)
_KS = (0, 1, 0x1BD11BDB)


def _rotl(x, r):
    return (x << jnp.uint32(r)) | (x >> jnp.uint32(32 - r))


def _threefry_bits(j):
    ks = tuple(jnp.uint32(k) for k in _KS)
    x1 = j + ks[1]
    x0 = x1
    x1 = _rotl(x1, _ROT_A[0]) ^ x0
    for r in _ROT_A[1:]:
        x0 = x0 + x1
        x1 = _rotl(x1, r)
        x1 = x0 ^ x1
    inject = ((ks[1], ks[2], 1), (ks[2], ks[0], 2), (ks[0], ks[1], 3),
              (ks[1], ks[2], 4), (ks[2], ks[0], 5))
    rots = (_ROT_B, _ROT_A, _ROT_B, _ROT_A)
    for (ka, kb, c), rgroup in zip(inject, rots + ((),)):
        x0 = x0 + ka
        x1 = x1 + kb + jnp.uint32(c)
        for r in rgroup:
            x0 = x0 + x1
            x1 = _rotl(x1, r)
            x1 = x0 ^ x1
    return x0 ^ x1


def _z_chunk(x, idx, rt):
    bits = _threefry_bits(idx)
    f = jax.lax.bitcast_convert_type(
        (bits >> jnp.uint32(9)) | jnp.uint32(0x3F800000), jnp.float32)
    u = f - jnp.float32(1.0)
    eps = jnp.float32(1e-20)
    g = -jnp.log(-jnp.log(u + eps) + eps)
    return (x + g) * rt


def _body(x_ref, t_ref, o_ref, *, block_rows):
    i = pl.program_id(0)
    rt = jnp.float32(1.0) / t_ref[0]
    shape = (block_rows, _CHUNK)
    row = jax.lax.broadcasted_iota(jnp.uint32, shape, 0)
    col = jax.lax.broadcasted_iota(jnp.uint32, shape, 1)
    base_row = (i * block_rows).astype(jnp.uint32)
    idx0 = (base_row + row) * jnp.uint32(_N_COLS) + col

    def p1(k, m):
        off = k * _CHUNK
        z = _z_chunk(x_ref[:, pl.ds(off, _CHUNK)],
                     idx0 + off.astype(jnp.uint32), rt)
        o_ref[:, pl.ds(off, _CHUNK)] = z
        return jnp.maximum(m, z)

    m_lanes = jax.lax.fori_loop(
        0, _NFULL, p1, jnp.full(shape, -jnp.inf, jnp.float32))
    z_tail = _z_chunk(x_ref[:, _TAIL_START:],
                      idx0[:, :_TAIL] + jnp.uint32(_TAIL_START), rt)
    o_ref[:, _TAIL_START:] = z_tail
    m = jnp.maximum(jnp.max(m_lanes, axis=-1, keepdims=True),
                    jnp.max(z_tail, axis=-1, keepdims=True))

    def p2(k, s):
        off = k * _CHUNK
        e = jnp.exp(o_ref[:, pl.ds(off, _CHUNK)] - m)
        o_ref[:, pl.ds(off, _CHUNK)] = e
        return s + e

    s_lanes = jax.lax.fori_loop(0, _NFULL, p2, jnp.zeros(shape, jnp.float32))
    e_tail = jnp.exp(z_tail - m)
    o_ref[:, _TAIL_START:] = e_tail
    s = (jnp.sum(s_lanes, axis=-1, keepdims=True)
         + jnp.sum(e_tail, axis=-1, keepdims=True))
    rs = jnp.float32(1.0) / s

    def p3(k, carry):
        off = k * _CHUNK
        o_ref[:, pl.ds(off, _CHUNK)] = o_ref[:, pl.ds(off, _CHUNK)] * rs
        return carry

    jax.lax.fori_loop(0, _NFULL, p3, 0)
    o_ref[:, _TAIL_START:] = e_tail * rs


def kernel(logits, temperature, use_gpu):
    del use_gpu
    block_rows = 8
    t = jnp.float32(temperature).reshape(1)
    return pl.pallas_call(
        functools.partial(_body, block_rows=block_rows),
        grid=(_N_ROWS // block_rows,),
        in_specs=[
            pl.BlockSpec((block_rows, _N_COLS), lambda i: (i, 0)),
            pl.BlockSpec(memory_space=pltpu.SMEM),
        ],
        out_specs=pl.BlockSpec((block_rows, _N_COLS), lambda i: (i, 0)),
        out_shape=jax.ShapeDtypeStruct((_N_ROWS, _N_COLS), jnp.float32),
    )(logits, t)


# speedup vs baseline: 1.5755x; 1.0015x over previous
import functools

import jax
import jax.numpy as jnp
from jax.experimental import pallas as pl
from jax.experimental.pallas import tpu as pltpu

_N_ROWS = 128
_N_COLS = 100000

_CHUNK = 8192
_NFULL = _N_COLS // _CHUNK
_TAIL_START = _NFULL * _CHUNK
_TAIL = _N_COLS - _TAIL_START

_ROT_A = (13, 15, 26, 6)
_ROT_B = (17, 29, 16, 24)
_KS = (0, 1, 0x1BD11BDB)


def _rotl(x, r):
    return (x << jnp.uint32(r)) | (x >> jnp.uint32(32 - r))


def _threefry_bits(j):
    ks = tuple(jnp.uint32(k) for k in _KS)
    x1 = j + ks[1]
    x0 = x1
    x1 = _rotl(x1, _ROT_A[0]) ^ x0
    for r in _ROT_A[1:]:
        x0 = x0 + x1
        x1 = _rotl(x1, r)
        x1 = x0 ^ x1
    inject = ((ks[1], ks[2], 1), (ks[2], ks[0], 2), (ks[0], ks[1], 3),
              (ks[1], ks[2], 4), (ks[2], ks[0], 5))
    rots = (_ROT_B, _ROT_A, _ROT_B, _ROT_A)
    for (ka, kb, c), rgroup in zip(inject, rots + ((),)):
        x0 = x0 + ka
        x1 = x1 + kb + jnp.uint32(c)
        for r in rgroup:
            x0 = x0 + x1
            x1 = _rotl(x1, r)
            x1 = x0 ^ x1
    return x0 ^ x1


def _z_chunk(x, idx, rt):
    bits = _threefry_bits(idx)
    f = jax.lax.bitcast_convert_type(
        (bits >> jnp.uint32(9)) | jnp.uint32(0x3F800000), jnp.float32)
    u = f - jnp.float32(1.0)
    eps = jnp.float32(1e-20)
    g = -jnp.log(-jnp.log(u + eps) + eps)
    return (x + g) * rt


def _body(x_ref, t_ref, o_ref, *, block_rows):
    i = pl.program_id(0)
    rt = jnp.float32(1.0) / t_ref[0].astype(jnp.float32)
    shape = (block_rows, _CHUNK)
    row = jax.lax.broadcasted_iota(jnp.uint32, shape, 0)
    col = jax.lax.broadcasted_iota(jnp.uint32, shape, 1)
    base_row = (i * block_rows).astype(jnp.uint32)
    idx0 = (base_row + row) * jnp.uint32(_N_COLS) + col

    def p1(k, m):
        off = k * _CHUNK
        z = _z_chunk(x_ref[:, pl.ds(off, _CHUNK)],
                     idx0 + off.astype(jnp.uint32), rt)
        o_ref[:, pl.ds(off, _CHUNK)] = z
        return jnp.maximum(m, z)

    m_lanes = jax.lax.fori_loop(
        0, _NFULL, p1, jnp.full(shape, -jnp.inf, jnp.float32))
    z_tail = _z_chunk(x_ref[:, _TAIL_START:],
                      idx0[:, :_TAIL] + jnp.uint32(_TAIL_START), rt)
    o_ref[:, _TAIL_START:] = z_tail
    m = jnp.maximum(jnp.max(m_lanes, axis=-1, keepdims=True),
                    jnp.max(z_tail, axis=-1, keepdims=True))

    def p2(k, s):
        off = k * _CHUNK
        e = jnp.exp(o_ref[:, pl.ds(off, _CHUNK)] - m)
        o_ref[:, pl.ds(off, _CHUNK)] = e
        return s + e

    s_lanes = jax.lax.fori_loop(0, _NFULL, p2, jnp.zeros(shape, jnp.float32))
    e_tail = jnp.exp(z_tail - m)
    o_ref[:, _TAIL_START:] = e_tail
    s = (jnp.sum(s_lanes, axis=-1, keepdims=True)
         + jnp.sum(e_tail, axis=-1, keepdims=True))
    rs = jnp.float32(1.0) / s

    def p3(k, carry):
        off = k * _CHUNK
        o_ref[:, pl.ds(off, _CHUNK)] = o_ref[:, pl.ds(off, _CHUNK)] * rs
        return carry

    jax.lax.fori_loop(0, _NFULL, p3, 0)
    o_ref[:, _TAIL_START:] = e_tail * rs


def kernel(logits, temperature, use_gpu):
    del use_gpu
    block_rows = 8
    t = jnp.reshape(temperature, (1,))
    return pl.pallas_call(
        functools.partial(_body, block_rows=block_rows),
        grid=(_N_ROWS // block_rows,),
        in_specs=[
            pl.BlockSpec((block_rows, _N_COLS), lambda i: (i, 0)),
            pl.BlockSpec(memory_space=pltpu.SMEM),
        ],
        out_specs=pl.BlockSpec((block_rows, _N_COLS), lambda i: (i, 0)),
        out_shape=jax.ShapeDtypeStruct((_N_ROWS, _N_COLS), jnp.float32),
    )(logits, t)
